# Initial kernel scaffold; baseline (speedup 1.0000x reference)
import jax
import jax.numpy as jnp
from jax.experimental import pallas as pl


def kernel(x, edge_index, edge_weight, W, b):
    raise NotImplementedError("write your pallas kernel here")


# speedup vs baseline: 55.8510x; 55.8510x over previous
import dataclasses
import functools

import jax
import jax.numpy as jnp
from jax import lax
from jax.experimental import pallas as pl
from jax.experimental.pallas import tpu as pltpu
from jax.experimental.pallas import tpu_sc as plsc

B, N, E, D = 2, 10000, 320000, 128
NC, NS = 2, 16
NPAD = 10240
EPS = E // NS
CHUNK = 80
NCH = EPS // CHUNK
ROWS_PER_TILE = NPAD // NS
ZROWS = ROWS_PER_TILE // 5
BN = 400
GB = N // BN

_vmesh = plsc.VectorSubcoreMesh(core_axis_name="c", subcore_axis_name="s")

_sc_params = pltpu.CompilerParams()
if "needs_layout_passes" in pltpu.CompilerParams.__dataclass_fields__:
    _sc_params = dataclasses.replace(_sc_params, needs_layout_passes=False)


NCH_A = E // (NC * NS * CHUNK)


@functools.partial(
    pl.kernel,
    out_type=jax.ShapeDtypeStruct((NC, NPAD, D), jnp.float32),
    mesh=_vmesh,
    scratch_types=[
        pltpu.VMEM((NCH_A, CHUNK), jnp.int32),
        pltpu.VMEM((ZROWS, D), jnp.float32),
        pltpu.VMEM_SHARED((NPAD, D), jnp.float32),
    ],
    compiler_params=_sc_params,
)
def _deg_kernel(dst_hbm, deg_hbm, dst_v, buf_v, deg_sh):
    c = lax.axis_index("c")
    s = lax.axis_index("s")

    pltpu.sync_copy(dst_hbm.at[c, s], dst_v)

    one = jnp.ones((16,), jnp.float32)
    zero = jnp.zeros((16,), jnp.float32)

    @pl.loop(0, ZROWS)
    def _(r):
        for k in range(D // 16):
            buf_v[r, pl.ds(16 * k, 16)] = zero

    for k in range(ROWS_PER_TILE // ZROWS):
        pltpu.sync_copy(
            buf_v,
            deg_sh.at[pl.ds(s * ROWS_PER_TILE + k * ZROWS, ZROWS)],
        )

    @pl.loop(0, CHUNK)
    def _(r):
        for k in range(D // 16):
            buf_v[r, pl.ds(16 * k, 16)] = one

    plsc.subcore_barrier()

    @pl.loop(0, NCH_A)
    def _(g):
        pltpu.sync_copy(buf_v.at[pl.ds(0, CHUNK)], deg_sh.at[dst_v.at[g]],
                        add=True)

    plsc.subcore_barrier()
    pltpu.sync_copy(
        deg_sh.at[pl.ds(s * ROWS_PER_TILE, ROWS_PER_TILE)],
        deg_hbm.at[c].at[pl.ds(s * ROWS_PER_TILE, ROWS_PER_TILE)],
    )


@functools.partial(
    pl.kernel,
    out_type=jax.ShapeDtypeStruct((B, NPAD, D), jnp.float32),
    mesh=_vmesh,
    scratch_types=[
        pltpu.VMEM((3, CHUNK), jnp.int32),
        pltpu.VMEM((3, CHUNK), jnp.int32),
        pltpu.VMEM((CHUNK, D), jnp.float32),
        pltpu.VMEM((CHUNK, D), jnp.float32),
        pltpu.VMEM_SHARED((NPAD, D), jnp.float32),
        pltpu.SemaphoreType.DMA,
        pltpu.SemaphoreType.DMA,
    ],
    compiler_params=_sc_params,
)
def _agg_kernel(ys_hbm, edata_hbm, acc_hbm,
                ebuf0, ebuf1, rows0, rows1, acc_sh, sem0, sem1):
    c = lax.axis_index("c")
    s = lax.axis_index("s")

    zero = jnp.zeros((16,), jnp.float32)

    @pl.loop(0, CHUNK)
    def _(r):
        for k in range(D // 16):
            rows0[r, pl.ds(16 * k, 16)] = zero

    for k in range(ROWS_PER_TILE // CHUNK):
        pltpu.sync_copy(
            rows0,
            acc_sh.at[pl.ds(s * ROWS_PER_TILE + k * CHUNK, CHUNK)],
        )
    plsc.subcore_barrier()

    def scale_scatter(ebuf, buf):
        @pl.loop(0, CHUNK)
        def _(r):
            wspl = plsc.bitcast(
                plsc.load_gather(
                    ebuf,
                    [jnp.full((16,), 2, jnp.int32), jnp.broadcast_to(r, (16,))],
                ),
                jnp.float32,
            )
            for k in range(D // 16):
                slc = pl.ds(16 * k, 16)
                buf[r, slc] = buf[r, slc] * wspl

        pltpu.sync_copy(buf, acc_sh.at[ebuf.at[1]], add=True)

    @pl.loop(0, NCH // 2)
    def _(i):
        e = 2 * i
        o = e + 1
        pltpu.sync_copy(edata_hbm.at[c, s, e], ebuf0)
        d0 = pltpu.async_copy(ys_hbm.at[ebuf0.at[0]], rows0, sem0)
        pltpu.sync_copy(edata_hbm.at[c, s, o], ebuf1)
        d1 = pltpu.async_copy(ys_hbm.at[ebuf1.at[0]], rows1, sem1)
        d0.wait()
        scale_scatter(ebuf0, rows0)
        d1.wait()
        scale_scatter(ebuf1, rows1)

    plsc.subcore_barrier()
    pltpu.sync_copy(
        acc_sh.at[pl.ds(s * ROWS_PER_TILE, ROWS_PER_TILE)],
        acc_hbm.at[c].at[pl.ds(s * ROWS_PER_TILE, ROWS_PER_TILE)],
    )


def _pre_body(x_ref, deg0_ref, deg1_ref, w_ref, ys_ref, dis_ref):
    deg = deg0_ref[0] + deg1_ref[0]
    dis = lax.rsqrt(jnp.maximum(deg, 1.0))
    y = lax.dot_general(x_ref[...], w_ref[...], (((1,), (1,)), ((), ())),
                        preferred_element_type=jnp.float32)
    ys_ref[...] = y * dis
    dis_ref[...] = dis


_pre_call = pl.pallas_call(
    _pre_body,
    grid=(B, GB),
    in_specs=[
        pl.BlockSpec((BN, D), lambda b, j: (b * GB + j, 0)),
        pl.BlockSpec((1, BN, D), lambda b, j: (0, j, 0)),
        pl.BlockSpec((1, BN, D), lambda b, j: (1, j, 0)),
        pl.BlockSpec((D, D), lambda b, j: (0, 0)),
    ],
    out_specs=[
        pl.BlockSpec((BN, D), lambda b, j: (b * GB + j, 0)),
        pl.BlockSpec((BN, D), lambda b, j: (j, 0)),
    ],
    out_shape=[
        jax.ShapeDtypeStruct((B * N, D), jnp.float32),
        jax.ShapeDtypeStruct((N, D), jnp.float32),
    ],
)


def _post_body(acc_ref, dis_ref, b_ref, out_ref):
    out_ref[...] = (acc_ref[...] * dis_ref[...][None, :, :]
                    + b_ref[0:1, :][None, :, :])


_post_call = pl.pallas_call(
    _post_body,
    grid=(B, GB),
    in_specs=[
        pl.BlockSpec((1, BN, D), lambda b, j: (b, j, 0)),
        pl.BlockSpec((BN, D), lambda b, j: (j, 0)),
        pl.BlockSpec((8, D), lambda b, j: (0, 0)),
    ],
    out_specs=pl.BlockSpec((1, BN, D), lambda b, j: (b, j, 0)),
    out_shape=jax.ShapeDtypeStruct((B, N, D), jnp.float32),
)


@jax.jit
def _run(x, edge_index, edge_weight, W, b):
    src4 = edge_index[0].reshape(NS, NCH, 1, CHUNK)
    dst4 = edge_index[1].reshape(NS, NCH, 1, CHUNK)
    w4 = lax.bitcast_convert_type(edge_weight, jnp.int32).reshape(
        NS, NCH, 1, CHUNK)
    edata1 = jnp.concatenate([src4, dst4, w4], axis=2)
    shift = jnp.array([N, 0, 0], jnp.int32).reshape(1, 1, 3, 1)
    edata = jnp.stack([edata1, edata1 + shift])
    dst4a = edge_index[1].reshape(NC, NS, NCH_A, CHUNK)
    x2 = x.reshape(B * N, D)
    bias8 = jnp.broadcast_to(b[None, :], (8, D))

    degp = _deg_kernel(dst4a)
    ys, dis128 = _pre_call(x2, degp, degp, W)
    acc = _agg_kernel(ys, edata)
    return _post_call(acc, dis128, bias8)


def kernel(x, edge_index, edge_weight, W, b):
    return _run(x, edge_index, edge_weight, W, b)


# speedup vs baseline: 65.9113x; 1.1801x over previous
import dataclasses
import functools

import jax
import jax.numpy as jnp
from jax import lax
from jax.experimental import pallas as pl
from jax.experimental.pallas import tpu as pltpu
from jax.experimental.pallas import tpu_sc as plsc

B, N, E, D = 2, 10000, 320000, 128
NC, NS = 2, 16
NPAD = 10240
EPS = E // NS
CHUNK = 80
NCH = EPS // CHUNK
ROWS_PER_TILE = NPAD // NS
ZROWS = ROWS_PER_TILE // 5
BN = 400
GB = N // BN

_vmesh = plsc.VectorSubcoreMesh(core_axis_name="c", subcore_axis_name="s")

_sc_params = pltpu.CompilerParams()
if "needs_layout_passes" in pltpu.CompilerParams.__dataclass_fields__:
    _sc_params = dataclasses.replace(_sc_params, needs_layout_passes=False)


NCH_A = E // (NC * NS * CHUNK)


@functools.partial(
    pl.kernel,
    out_type=jax.ShapeDtypeStruct((NC, NPAD, D), jnp.float32),
    mesh=_vmesh,
    scratch_types=[
        pltpu.VMEM((NCH_A, CHUNK), jnp.int32),
        pltpu.VMEM((ZROWS, D), jnp.float32),
        pltpu.VMEM_SHARED((NPAD, D), jnp.float32),
    ],
    compiler_params=_sc_params,
)
def _deg_kernel(dst_hbm, deg_hbm, dst_v, buf_v, deg_sh):
    c = lax.axis_index("c")
    s = lax.axis_index("s")

    pltpu.sync_copy(dst_hbm.at[c, s], dst_v)

    one = jnp.ones((16,), jnp.float32)
    zero = jnp.zeros((16,), jnp.float32)

    @pl.loop(0, ZROWS)
    def _(r):
        for k in range(D // 16):
            buf_v[r, pl.ds(16 * k, 16)] = zero

    for k in range(ROWS_PER_TILE // ZROWS):
        pltpu.sync_copy(
            buf_v,
            deg_sh.at[pl.ds(s * ROWS_PER_TILE + k * ZROWS, ZROWS)],
        )

    @pl.loop(0, CHUNK)
    def _(r):
        for k in range(D // 16):
            buf_v[r, pl.ds(16 * k, 16)] = one

    plsc.subcore_barrier()

    @pl.loop(0, NCH_A)
    def _(g):
        pltpu.sync_copy(buf_v.at[pl.ds(0, CHUNK)], deg_sh.at[dst_v.at[g]],
                        add=True)

    plsc.subcore_barrier()
    pltpu.sync_copy(
        deg_sh.at[pl.ds(s * ROWS_PER_TILE, ROWS_PER_TILE)],
        deg_hbm.at[c].at[pl.ds(s * ROWS_PER_TILE, ROWS_PER_TILE)],
    )


@functools.partial(
    pl.kernel,
    out_type=jax.ShapeDtypeStruct((B, NPAD, D), jnp.float32),
    mesh=_vmesh,
    scratch_types=[
        pltpu.VMEM((3, CHUNK), jnp.int32),
        pltpu.VMEM((3, CHUNK), jnp.int32),
        pltpu.VMEM((CHUNK, D), jnp.float32),
        pltpu.VMEM((CHUNK, D), jnp.float32),
        pltpu.VMEM_SHARED((NPAD, D), jnp.float32),
        pltpu.SemaphoreType.DMA,
        pltpu.SemaphoreType.DMA,
    ],
    compiler_params=_sc_params,
)
def _agg_kernel(ys_hbm, edata_hbm, acc_hbm,
                ebuf0, ebuf1, rows0, rows1, acc_sh, sem0, sem1):
    c = lax.axis_index("c")
    s = lax.axis_index("s")

    zero = jnp.zeros((16,), jnp.float32)

    @pl.loop(0, CHUNK)
    def _(r):
        for k in range(D // 16):
            rows0[r, pl.ds(16 * k, 16)] = zero

    for k in range(ROWS_PER_TILE // CHUNK):
        pltpu.sync_copy(
            rows0,
            acc_sh.at[pl.ds(s * ROWS_PER_TILE + k * CHUNK, CHUNK)],
        )
    plsc.subcore_barrier()

    def scale_scatter(ebuf, buf):
        @pl.loop(0, CHUNK, step=16)
        def _(r0):
            for j in range(16):
                wspl = plsc.bitcast(
                    plsc.load_gather(
                        ebuf,
                        [jnp.full((16,), 2, jnp.int32),
                         jnp.broadcast_to(r0 + j, (16,))],
                    ),
                    jnp.float32,
                )
                for k in range(D // 16):
                    slc = pl.ds(16 * k, 16)
                    buf[r0 + j, slc] = buf[r0 + j, slc] * wspl

        pass

    @pl.loop(0, NCH // 2)
    def _(i):
        e = 2 * i
        o = e + 1
        pltpu.sync_copy(edata_hbm.at[c, s, e], ebuf0)
        d0 = pltpu.async_copy(ys_hbm.at[ebuf0.at[0]], rows0, sem0)
        pltpu.sync_copy(edata_hbm.at[c, s, o], ebuf1)
        d1 = pltpu.async_copy(ys_hbm.at[ebuf1.at[0]], rows1, sem1)
        d0.wait()
        scale_scatter(ebuf0, rows0)
        d1.wait()
        scale_scatter(ebuf1, rows1)

    plsc.subcore_barrier()
    pltpu.sync_copy(
        acc_sh.at[pl.ds(s * ROWS_PER_TILE, ROWS_PER_TILE)],
        acc_hbm.at[c].at[pl.ds(s * ROWS_PER_TILE, ROWS_PER_TILE)],
    )


def _mm_body(x_ref, w_ref, y_ref):
    y_ref[...] = lax.dot_general(x_ref[...], w_ref[...],
                                 (((1,), (1,)), ((), ())),
                                 preferred_element_type=jnp.float32)


_mm_call = pl.pallas_call(
    _mm_body,
    grid=(B * GB,),
    in_specs=[
        pl.BlockSpec((BN, D), lambda j: (j, 0)),
        pl.BlockSpec((D, D), lambda j: (0, 0)),
    ],
    out_specs=pl.BlockSpec((BN, D), lambda j: (j, 0)),
    out_shape=jax.ShapeDtypeStruct((B * N, D), jnp.float32),
)


def _pre_body(y_ref, deg0_ref, deg1_ref, ys_ref, dis_ref):
    deg = deg0_ref[0] + deg1_ref[0]
    dis = lax.rsqrt(jnp.maximum(deg, 1.0))
    ys_ref[...] = y_ref[...] * dis
    dis_ref[...] = dis


_pre_call = pl.pallas_call(
    _pre_body,
    grid=(B, GB),
    in_specs=[
        pl.BlockSpec((BN, D), lambda b, j: (b * GB + j, 0)),
        pl.BlockSpec((1, BN, D), lambda b, j: (0, j, 0)),
        pl.BlockSpec((1, BN, D), lambda b, j: (1, j, 0)),
    ],
    out_specs=[
        pl.BlockSpec((BN, D), lambda b, j: (b * GB + j, 0)),
        pl.BlockSpec((BN, D), lambda b, j: (j, 0)),
    ],
    out_shape=[
        jax.ShapeDtypeStruct((B * N, D), jnp.float32),
        jax.ShapeDtypeStruct((N, D), jnp.float32),
    ],
)


def _post_body(acc_ref, dis_ref, b_ref, out_ref):
    out_ref[...] = (acc_ref[...] * dis_ref[...][None, :, :]
                    + b_ref[0:1, :][None, :, :])


_post_call = pl.pallas_call(
    _post_body,
    grid=(B, GB),
    in_specs=[
        pl.BlockSpec((1, BN, D), lambda b, j: (b, j, 0)),
        pl.BlockSpec((BN, D), lambda b, j: (j, 0)),
        pl.BlockSpec((8, D), lambda b, j: (0, 0)),
    ],
    out_specs=pl.BlockSpec((1, BN, D), lambda b, j: (b, j, 0)),
    out_shape=jax.ShapeDtypeStruct((B, N, D), jnp.float32),
)


@jax.jit
def _run(x, edge_index, edge_weight, W, b):
    src4 = edge_index[0].reshape(NS, NCH, 1, CHUNK)
    dst4 = edge_index[1].reshape(NS, NCH, 1, CHUNK)
    w4 = lax.bitcast_convert_type(edge_weight, jnp.int32).reshape(
        NS, NCH, 1, CHUNK)
    edata1 = jnp.concatenate([src4, dst4, w4], axis=2)
    shift = jnp.array([N, 0, 0], jnp.int32).reshape(1, 1, 3, 1)
    edata = jnp.stack([edata1, edata1 + shift])
    dst4a = edge_index[1].reshape(NC, NS, NCH_A, CHUNK)
    x2 = x.reshape(B * N, D)
    bias8 = jnp.broadcast_to(b[None, :], (8, D))

    degp = _deg_kernel(dst4a)
    y = _mm_call(x2, W)
    ys, dis128 = _pre_call(y, degp, degp)
    acc = _agg_kernel(ys, edata)
    return _post_call(acc, dis128, bias8)


def kernel(x, edge_index, edge_weight, W, b):
    return _run(x, edge_index, edge_weight, W, b)


# speedup vs baseline: 73.0215x; 1.1079x over previous
import dataclasses
import functools

import jax
import jax.numpy as jnp
from jax import lax
from jax.experimental import pallas as pl
from jax.experimental.pallas import tpu as pltpu
from jax.experimental.pallas import tpu_sc as plsc

B, N, E, D = 2, 10000, 320000, 128
NC, NS = 2, 16
NPAD = 10240
EPS = E // NS
CHUNK = 80
NCH = EPS // CHUNK
ROWS_PER_TILE = NPAD // NS
ZROWS = ROWS_PER_TILE // 5
BN = 400
GB = N // BN

_vmesh = plsc.VectorSubcoreMesh(core_axis_name="c", subcore_axis_name="s")

_sc_params = pltpu.CompilerParams()
if "needs_layout_passes" in pltpu.CompilerParams.__dataclass_fields__:
    _sc_params = dataclasses.replace(_sc_params, needs_layout_passes=False)


NCH_A = E // (NC * NS * CHUNK)


@functools.partial(
    pl.kernel,
    out_type=jax.ShapeDtypeStruct((NC, NPAD, D), jnp.float32),
    mesh=_vmesh,
    scratch_types=[
        pltpu.VMEM((NCH_A, CHUNK), jnp.int32),
        pltpu.VMEM((ZROWS, D), jnp.float32),
        pltpu.VMEM_SHARED((NPAD, D), jnp.float32),
    ],
    compiler_params=_sc_params,
)
def _deg_kernel(dst_hbm, deg_hbm, dst_v, buf_v, deg_sh):
    c = lax.axis_index("c")
    s = lax.axis_index("s")

    pltpu.sync_copy(dst_hbm.at[c, s], dst_v)

    one = jnp.ones((16,), jnp.float32)
    zero = jnp.zeros((16,), jnp.float32)

    @pl.loop(0, ZROWS)
    def _(r):
        for k in range(D // 16):
            buf_v[r, pl.ds(16 * k, 16)] = zero

    for k in range(ROWS_PER_TILE // ZROWS):
        pltpu.sync_copy(
            buf_v,
            deg_sh.at[pl.ds(s * ROWS_PER_TILE + k * ZROWS, ZROWS)],
        )

    @pl.loop(0, CHUNK)
    def _(r):
        for k in range(D // 16):
            buf_v[r, pl.ds(16 * k, 16)] = one

    plsc.subcore_barrier()

    @pl.loop(0, NCH_A)
    def _(g):
        pltpu.sync_copy(buf_v.at[pl.ds(0, CHUNK)], deg_sh.at[dst_v.at[g]],
                        add=True)

    plsc.subcore_barrier()
    pltpu.sync_copy(
        deg_sh.at[pl.ds(s * ROWS_PER_TILE, ROWS_PER_TILE)],
        deg_hbm.at[c].at[pl.ds(s * ROWS_PER_TILE, ROWS_PER_TILE)],
    )


NPAIR = NCH // 2


@functools.partial(
    pl.kernel,
    out_type=jax.ShapeDtypeStruct((B, NPAD, D), jnp.float32),
    mesh=_vmesh,
    scratch_types=[
        pltpu.VMEM((3, CHUNK), jnp.int32),
        pltpu.VMEM((3, CHUNK), jnp.int32),
        pltpu.VMEM((3, CHUNK), jnp.int32),
        pltpu.VMEM((3, CHUNK), jnp.int32),
        pltpu.VMEM((CHUNK, D), jnp.float32),
        pltpu.VMEM((CHUNK, D), jnp.float32),
        pltpu.VMEM_SHARED((NPAD, D), jnp.float32),
        pltpu.SemaphoreType.DMA,
        pltpu.SemaphoreType.DMA,
        pltpu.SemaphoreType.DMA,
    ],
    compiler_params=_sc_params,
)
def _agg_kernel(ys_hbm, edata_hbm, acc_hbm,
                ebufA0, ebufA1, ebufB0, ebufB1,
                rows0, rows1, acc_sh, gsemA, gsemB, esem):
    c = lax.axis_index("c")
    s = lax.axis_index("s")

    zero = jnp.zeros((16,), jnp.float32)

    @pl.loop(0, CHUNK)
    def _(r):
        for k in range(D // 16):
            rows0[r, pl.ds(16 * k, 16)] = zero

    for k in range(ROWS_PER_TILE // CHUNK):
        pltpu.sync_copy(
            rows0,
            acc_sh.at[pl.ds(s * ROWS_PER_TILE + k * CHUNK, CHUNK)],
        )
    plsc.subcore_barrier()

    def scale(ebuf, buf):
        @pl.loop(0, CHUNK, step=16)
        def _(r0):
            for jj in range(16):
                wspl = plsc.bitcast(
                    plsc.load_gather(
                        ebuf,
                        [jnp.full((16,), 2, jnp.int32),
                         jnp.broadcast_to(r0 + jj, (16,))],
                    ),
                    jnp.float32,
                )
                for k in range(D // 16):
                    slc = pl.ds(16 * k, 16)
                    buf[r0 + jj, slc] = buf[r0 + jj, slc] * wspl

    def issue_gather(ebuf, buf, sem):
        return pltpu.async_copy(ys_hbm.at[ebuf.at[0]], buf, sem)

    def wait_gather(ebuf, buf, sem):
        pltpu.make_async_copy(ys_hbm.at[ebuf.at[0]], buf, sem).wait()

    def wait_stage(ebuf):
        pltpu.make_async_copy(edata_hbm.at[c, s, 0], ebuf, esem).wait()

    pltpu.sync_copy(edata_hbm.at[c, s, 0], ebufA0)
    pltpu.sync_copy(edata_hbm.at[c, s, 1], ebufA1)
    issue_gather(ebufA0, rows0, gsemA)
    issue_gather(ebufA1, rows1, gsemB)
    pltpu.async_copy(edata_hbm.at[c, s, 2], ebufB0, esem)
    pltpu.async_copy(edata_hbm.at[c, s, 3], ebufB1, esem)

    def pair_body(eb0, eb1, nb0, nb1, i):
        wait_gather(eb0, rows0, gsemA)
        scale(eb0, rows0)
        dA = pltpu.async_copy(rows0, acc_sh.at[eb0.at[1]], gsemA, add=True)
        wait_gather(eb1, rows1, gsemB)
        scale(eb1, rows1)
        dB = pltpu.async_copy(rows1, acc_sh.at[eb1.at[1]], gsemB, add=True)
        dA.wait()

        @pl.when(i < NPAIR - 1)
        def _():
            wait_stage(nb0)
            wait_stage(nb1)
            issue_gather(nb0, rows0, gsemA)

        dB.wait()

        @pl.when(i < NPAIR - 1)
        def _():
            issue_gather(nb1, rows1, gsemB)

            @pl.when(i < NPAIR - 2)
            def _():
                pltpu.async_copy(edata_hbm.at[c, s, 2 * i + 4], eb0, esem)
                pltpu.async_copy(edata_hbm.at[c, s, 2 * i + 5], eb1, esem)

    @pl.loop(0, NPAIR)
    def _(i):
        parity = lax.rem(i, 2)

        @pl.when(parity == 0)
        def _():
            pair_body(ebufA0, ebufA1, ebufB0, ebufB1, i)

        @pl.when(parity == 1)
        def _():
            pair_body(ebufB0, ebufB1, ebufA0, ebufA1, i)

    plsc.subcore_barrier()
    pltpu.sync_copy(
        acc_sh.at[pl.ds(s * ROWS_PER_TILE, ROWS_PER_TILE)],
        acc_hbm.at[c].at[pl.ds(s * ROWS_PER_TILE, ROWS_PER_TILE)],
    )


def _mm_body(x_ref, w_ref, y_ref):
    y_ref[...] = lax.dot_general(x_ref[...], w_ref[...],
                                 (((1,), (1,)), ((), ())),
                                 preferred_element_type=jnp.float32)


_mm_call = pl.pallas_call(
    _mm_body,
    grid=(B * GB,),
    in_specs=[
        pl.BlockSpec((BN, D), lambda j: (j, 0)),
        pl.BlockSpec((D, D), lambda j: (0, 0)),
    ],
    out_specs=pl.BlockSpec((BN, D), lambda j: (j, 0)),
    out_shape=jax.ShapeDtypeStruct((B * N, D), jnp.float32),
)


def _pre_body(y_ref, deg0_ref, deg1_ref, ys_ref, dis_ref):
    deg = deg0_ref[0] + deg1_ref[0]
    dis = lax.rsqrt(jnp.maximum(deg, 1.0))
    ys_ref[...] = y_ref[...] * dis
    dis_ref[...] = dis


_pre_call = pl.pallas_call(
    _pre_body,
    grid=(B, GB),
    in_specs=[
        pl.BlockSpec((BN, D), lambda b, j: (b * GB + j, 0)),
        pl.BlockSpec((1, BN, D), lambda b, j: (0, j, 0)),
        pl.BlockSpec((1, BN, D), lambda b, j: (1, j, 0)),
    ],
    out_specs=[
        pl.BlockSpec((BN, D), lambda b, j: (b * GB + j, 0)),
        pl.BlockSpec((BN, D), lambda b, j: (j, 0)),
    ],
    out_shape=[
        jax.ShapeDtypeStruct((B * N, D), jnp.float32),
        jax.ShapeDtypeStruct((N, D), jnp.float32),
    ],
)


def _post_body(acc_ref, dis_ref, b_ref, out_ref):
    out_ref[...] = (acc_ref[...] * dis_ref[...][None, :, :]
                    + b_ref[0:1, :][None, :, :])


_post_call = pl.pallas_call(
    _post_body,
    grid=(B, GB),
    in_specs=[
        pl.BlockSpec((1, BN, D), lambda b, j: (b, j, 0)),
        pl.BlockSpec((BN, D), lambda b, j: (j, 0)),
        pl.BlockSpec((8, D), lambda b, j: (0, 0)),
    ],
    out_specs=pl.BlockSpec((1, BN, D), lambda b, j: (b, j, 0)),
    out_shape=jax.ShapeDtypeStruct((B, N, D), jnp.float32),
)


@jax.jit
def _run(x, edge_index, edge_weight, W, b):
    src4 = edge_index[0].reshape(NS, NCH, 1, CHUNK)
    dst4 = edge_index[1].reshape(NS, NCH, 1, CHUNK)
    w4 = lax.bitcast_convert_type(edge_weight, jnp.int32).reshape(
        NS, NCH, 1, CHUNK)
    edata1 = jnp.concatenate([src4, dst4, w4], axis=2)
    shift = jnp.array([N, 0, 0], jnp.int32).reshape(1, 1, 3, 1)
    edata = jnp.stack([edata1, edata1 + shift])
    dst4a = edge_index[1].reshape(NC, NS, NCH_A, CHUNK)
    x2 = x.reshape(B * N, D)
    bias8 = jnp.broadcast_to(b[None, :], (8, D))

    degp = _deg_kernel(dst4a)
    y = _mm_call(x2, W)
    ys, dis128 = _pre_call(y, degp, degp)
    acc = _agg_kernel(ys, edata)
    return _post_call(acc, dis128, bias8)


def kernel(x, edge_index, edge_weight, W, b):
    return _run(x, edge_index, edge_weight, W, b)


# speedup vs baseline: 73.0513x; 1.0004x over previous
import dataclasses
import functools

import jax
import jax.numpy as jnp
from jax import lax
from jax.experimental import pallas as pl
from jax.experimental.pallas import tpu as pltpu
from jax.experimental.pallas import tpu_sc as plsc

B, N, E, D = 2, 10000, 320000, 128
NC, NS = 2, 16
NPAD = 10240
EPS = E // NS
CHUNK = 80
NCH = EPS // CHUNK
ROWS_PER_TILE = NPAD // NS
ZROWS = ROWS_PER_TILE // 5
BN = 400
GB = N // BN

_vmesh = plsc.VectorSubcoreMesh(core_axis_name="c", subcore_axis_name="s")

_sc_params = pltpu.CompilerParams()
if "needs_layout_passes" in pltpu.CompilerParams.__dataclass_fields__:
    _sc_params = dataclasses.replace(_sc_params, needs_layout_passes=False)


NCH_A = E // (NC * NS * CHUNK)


@functools.partial(
    pl.kernel,
    out_type=jax.ShapeDtypeStruct((NC, NPAD, D), jnp.float32),
    mesh=_vmesh,
    scratch_types=[
        pltpu.VMEM((NCH_A, CHUNK), jnp.int32),
        pltpu.VMEM((NPAD,), jnp.float32),
        pltpu.VMEM((ROWS_PER_TILE,), jnp.float32),
        pltpu.VMEM((ROWS_PER_TILE,), jnp.float32),
        pltpu.VMEM((ZROWS, D), jnp.float32),
        pltpu.VMEM_SHARED((NS, NPAD), jnp.float32),
    ],
    compiler_params=_sc_params,
)
def _deg_kernel(dst_hbm, deg_hbm, dst_v, hist_v, tmp_v, red_v, big_v, hist_sh):
    c = lax.axis_index("c")
    s = lax.axis_index("s")

    pltpu.sync_copy(dst_hbm.at[c, s], dst_v)

    one = jnp.ones((16,), jnp.float32)
    zero = jnp.zeros((16,), jnp.float32)

    @pl.loop(0, NPAD // 16)
    def _(r):
        hist_v[pl.ds(16 * r, 16)] = zero

    @pl.loop(0, NCH_A)
    def _(g):
        for k in range(CHUNK // 16):
            idx = dst_v[g, pl.ds(16 * k, 16)]
            plsc.addupdate_scatter(hist_v, [idx], one)

    pltpu.sync_copy(hist_v, hist_sh.at[s])
    plsc.subcore_barrier()

    @pl.loop(0, ROWS_PER_TILE // 16)
    def _(r):
        red_v[pl.ds(16 * r, 16)] = zero

    for t in range(NS):
        pltpu.sync_copy(
            hist_sh.at[t].at[pl.ds(s * ROWS_PER_TILE, ROWS_PER_TILE)], tmp_v)

        @pl.loop(0, ROWS_PER_TILE // 16)
        def _(r):
            slc = pl.ds(16 * r, 16)
            red_v[slc] = red_v[slc] + tmp_v[slc]

    for k5 in range(ROWS_PER_TILE // ZROWS):
        @pl.loop(0, ZROWS)
        def _(r):
            wspl = plsc.load_gather(
                red_v, [jnp.broadcast_to(k5 * ZROWS + r, (16,))])
            for kk in range(D // 16):
                big_v[r, pl.ds(16 * kk, 16)] = wspl

        pltpu.sync_copy(
            big_v,
            deg_hbm.at[c].at[pl.ds(s * ROWS_PER_TILE + k5 * ZROWS, ZROWS)],
        )


NPAIR = NCH // 2


@functools.partial(
    pl.kernel,
    out_type=jax.ShapeDtypeStruct((B, NPAD, D), jnp.float32),
    mesh=_vmesh,
    scratch_types=[
        pltpu.VMEM((3, CHUNK), jnp.int32),
        pltpu.VMEM((3, CHUNK), jnp.int32),
        pltpu.VMEM((3, CHUNK), jnp.int32),
        pltpu.VMEM((3, CHUNK), jnp.int32),
        pltpu.VMEM((CHUNK, D), jnp.float32),
        pltpu.VMEM((CHUNK, D), jnp.float32),
        pltpu.VMEM_SHARED((NPAD, D), jnp.float32),
        pltpu.SemaphoreType.DMA,
        pltpu.SemaphoreType.DMA,
        pltpu.SemaphoreType.DMA,
    ],
    compiler_params=_sc_params,
)
def _agg_kernel(ys_hbm, edata_hbm, acc_hbm,
                ebufA0, ebufA1, ebufB0, ebufB1,
                rows0, rows1, acc_sh, gsemA, gsemB, esem):
    c = lax.axis_index("c")
    s = lax.axis_index("s")

    zero = jnp.zeros((16,), jnp.float32)

    @pl.loop(0, CHUNK)
    def _(r):
        for k in range(D // 16):
            rows0[r, pl.ds(16 * k, 16)] = zero

    for k in range(ROWS_PER_TILE // CHUNK):
        pltpu.sync_copy(
            rows0,
            acc_sh.at[pl.ds(s * ROWS_PER_TILE + k * CHUNK, CHUNK)],
        )
    plsc.subcore_barrier()

    def scale(ebuf, buf):
        @pl.loop(0, CHUNK, step=16)
        def _(r0):
            for jj in range(16):
                wspl = plsc.bitcast(
                    plsc.load_gather(
                        ebuf,
                        [jnp.full((16,), 2, jnp.int32),
                         jnp.broadcast_to(r0 + jj, (16,))],
                    ),
                    jnp.float32,
                )
                for k in range(D // 16):
                    slc = pl.ds(16 * k, 16)
                    buf[r0 + jj, slc] = buf[r0 + jj, slc] * wspl

    def issue_gather(ebuf, buf, sem):
        return pltpu.async_copy(ys_hbm.at[ebuf.at[0]], buf, sem)

    def wait_gather(ebuf, buf, sem):
        pltpu.make_async_copy(ys_hbm.at[ebuf.at[0]], buf, sem).wait()

    def wait_stage(ebuf):
        pltpu.make_async_copy(edata_hbm.at[c, s, 0], ebuf, esem).wait()

    pltpu.sync_copy(edata_hbm.at[c, s, 0], ebufA0)
    pltpu.sync_copy(edata_hbm.at[c, s, 1], ebufA1)
    issue_gather(ebufA0, rows0, gsemA)
    issue_gather(ebufA1, rows1, gsemB)
    pltpu.async_copy(edata_hbm.at[c, s, 2], ebufB0, esem)
    pltpu.async_copy(edata_hbm.at[c, s, 3], ebufB1, esem)

    def pair_body(eb0, eb1, nb0, nb1, i):
        wait_gather(eb0, rows0, gsemA)
        scale(eb0, rows0)
        dA = pltpu.async_copy(rows0, acc_sh.at[eb0.at[1]], gsemA, add=True)
        wait_gather(eb1, rows1, gsemB)
        scale(eb1, rows1)
        dB = pltpu.async_copy(rows1, acc_sh.at[eb1.at[1]], gsemB, add=True)
        dA.wait()

        @pl.when(i < NPAIR - 1)
        def _():
            wait_stage(nb0)
            wait_stage(nb1)
            issue_gather(nb0, rows0, gsemA)

        dB.wait()

        @pl.when(i < NPAIR - 1)
        def _():
            issue_gather(nb1, rows1, gsemB)

            @pl.when(i < NPAIR - 2)
            def _():
                pltpu.async_copy(edata_hbm.at[c, s, 2 * i + 4], eb0, esem)
                pltpu.async_copy(edata_hbm.at[c, s, 2 * i + 5], eb1, esem)

    @pl.loop(0, NPAIR)
    def _(i):
        parity = lax.rem(i, 2)

        @pl.when(parity == 0)
        def _():
            pair_body(ebufA0, ebufA1, ebufB0, ebufB1, i)

        @pl.when(parity == 1)
        def _():
            pair_body(ebufB0, ebufB1, ebufA0, ebufA1, i)

    plsc.subcore_barrier()
    pltpu.sync_copy(
        acc_sh.at[pl.ds(s * ROWS_PER_TILE, ROWS_PER_TILE)],
        acc_hbm.at[c].at[pl.ds(s * ROWS_PER_TILE, ROWS_PER_TILE)],
    )


def _mm_body(x_ref, w_ref, y_ref):
    y_ref[...] = lax.dot_general(x_ref[...], w_ref[...],
                                 (((1,), (1,)), ((), ())),
                                 preferred_element_type=jnp.float32)


_mm_call = pl.pallas_call(
    _mm_body,
    grid=(B * GB,),
    in_specs=[
        pl.BlockSpec((BN, D), lambda j: (j, 0)),
        pl.BlockSpec((D, D), lambda j: (0, 0)),
    ],
    out_specs=pl.BlockSpec((BN, D), lambda j: (j, 0)),
    out_shape=jax.ShapeDtypeStruct((B * N, D), jnp.float32),
)


def _pre_body(y_ref, deg0_ref, deg1_ref, ys_ref, dis_ref):
    deg = deg0_ref[0] + deg1_ref[0]
    dis = lax.rsqrt(jnp.maximum(deg, 1.0))
    ys_ref[...] = y_ref[...] * dis
    dis_ref[...] = dis


_pre_call = pl.pallas_call(
    _pre_body,
    grid=(B, GB),
    in_specs=[
        pl.BlockSpec((BN, D), lambda b, j: (b * GB + j, 0)),
        pl.BlockSpec((1, BN, D), lambda b, j: (0, j, 0)),
        pl.BlockSpec((1, BN, D), lambda b, j: (1, j, 0)),
    ],
    out_specs=[
        pl.BlockSpec((BN, D), lambda b, j: (b * GB + j, 0)),
        pl.BlockSpec((BN, D), lambda b, j: (j, 0)),
    ],
    out_shape=[
        jax.ShapeDtypeStruct((B * N, D), jnp.float32),
        jax.ShapeDtypeStruct((N, D), jnp.float32),
    ],
)


def _post_body(acc_ref, dis_ref, b_ref, out_ref):
    out_ref[...] = (acc_ref[...] * dis_ref[...][None, :, :]
                    + b_ref[0:1, :][None, :, :])


_post_call = pl.pallas_call(
    _post_body,
    grid=(B, GB),
    in_specs=[
        pl.BlockSpec((1, BN, D), lambda b, j: (b, j, 0)),
        pl.BlockSpec((BN, D), lambda b, j: (j, 0)),
        pl.BlockSpec((8, D), lambda b, j: (0, 0)),
    ],
    out_specs=pl.BlockSpec((1, BN, D), lambda b, j: (b, j, 0)),
    out_shape=jax.ShapeDtypeStruct((B, N, D), jnp.float32),
)


@jax.jit
def _run(x, edge_index, edge_weight, W, b):
    src4 = edge_index[0].reshape(NS, NCH, 1, CHUNK)
    dst4 = edge_index[1].reshape(NS, NCH, 1, CHUNK)
    w4 = lax.bitcast_convert_type(edge_weight, jnp.int32).reshape(
        NS, NCH, 1, CHUNK)
    edata1 = jnp.concatenate([src4, dst4, w4], axis=2)
    shift = jnp.array([N, 0, 0], jnp.int32).reshape(1, 1, 3, 1)
    edata = jnp.stack([edata1, edata1 + shift])
    dst4a = edge_index[1].reshape(NC, NS, NCH_A, CHUNK)
    x2 = x.reshape(B * N, D)
    bias8 = jnp.broadcast_to(b[None, :], (8, D))

    degp = _deg_kernel(dst4a)
    y = _mm_call(x2, W)
    ys, dis128 = _pre_call(y, degp, degp)
    acc = _agg_kernel(ys, edata)
    return _post_call(acc, dis128, bias8)


def kernel(x, edge_index, edge_weight, W, b):
    return _run(x, edge_index, edge_weight, W, b)
